# Initial kernel scaffold; baseline (speedup 1.0000x reference)
import jax
import jax.numpy as jnp
from jax.experimental import pallas as pl


def kernel(x, edge_index, params):
    raise NotImplementedError("write your pallas kernel here")


# speedup vs baseline: 4.4677x; 4.4677x over previous
import functools

import jax
import jax.numpy as jnp
from jax import lax
from jax.experimental import pallas as pl
from jax.experimental.pallas import tpu as pltpu
from jax.experimental.pallas import tpu_sc as plsc

N = 10000
E = 320000
H = 128
L = 4

NC = 2
NS = 16
NWORK = NC * NS

W = 128
PER_TILE = E // NWORK
NWIN = -(-PER_TILE // W)
P = NWIN * W
TRASH = N
NACC = 10112
RPT = NACC // NS
CW = 128


def _seg_body(u_hbm, srcw_hbm, dstw_hbm, zeros_hbm, out_hbm,
              src_v, dst_v, buf_v, acc_sh):
    c = lax.axis_index("c")
    s = lax.axis_index("s")
    pltpu.sync_copy(srcw_hbm.at[c].at[s], src_v)
    pltpu.sync_copy(dstw_hbm.at[c].at[s], dst_v)
    pltpu.sync_copy(zeros_hbm.at[pl.ds(s * RPT, RPT)],
                    acc_sh.at[pl.ds(s * RPT, RPT)])
    plsc.subcore_barrier()

    def step(j, carry):
        pltpu.sync_copy(u_hbm.at[src_v.at[j]], buf_v)
        pltpu.sync_copy(buf_v, acc_sh.at[dst_v.at[j]], add=True)
        return carry

    lax.fori_loop(0, NWIN, step, 0)
    plsc.subcore_barrier()
    pltpu.sync_copy(acc_sh.at[pl.ds(s * RPT, RPT)],
                    out_hbm.at[c].at[pl.ds(s * RPT, RPT)])


@functools.cache
def _get_seg_kernel():
    return pl.kernel(
        _seg_body,
        out_type=jax.ShapeDtypeStruct((NC, NACC, H), jnp.float32),
        mesh=plsc.VectorSubcoreMesh(core_axis_name="c", subcore_axis_name="s",
                                    num_cores=NC, num_subcores=NS),
        scratch_types=[
            pltpu.VMEM((NWIN, W), jnp.int32),
            pltpu.VMEM((NWIN, W), jnp.int32),
            pltpu.VMEM((W, H), jnp.float32),
            pltpu.VMEM_SHARED((NACC, H), jnp.float32),
        ],
    )


def _seg_kernel(u, srcw, dstw, zeros):
    return _get_seg_kernel()(u, srcw, dstw, zeros)


def _cnt_body(dstw_hbm, ones_hbm, zeros_hbm, out_hbm, dst_v, ones_v, acc_sh):
    c = lax.axis_index("c")
    s = lax.axis_index("s")
    pltpu.sync_copy(dstw_hbm.at[c].at[s], dst_v)
    pltpu.sync_copy(ones_hbm, ones_v)
    pltpu.sync_copy(zeros_hbm.at[pl.ds(s * RPT, RPT)],
                    acc_sh.at[pl.ds(s * RPT, RPT)])
    plsc.subcore_barrier()

    def step(j, carry):
        pltpu.sync_copy(ones_v, acc_sh.at[dst_v.at[j]], add=True)
        return carry

    lax.fori_loop(0, NWIN, step, 0)
    plsc.subcore_barrier()
    pltpu.sync_copy(acc_sh.at[pl.ds(s * RPT, RPT)],
                    out_hbm.at[c].at[pl.ds(s * RPT, RPT)])


@functools.cache
def _get_cnt_kernel():
    return pl.kernel(
        _cnt_body,
        out_type=jax.ShapeDtypeStruct((NC, NACC, CW), jnp.float32),
        mesh=plsc.VectorSubcoreMesh(core_axis_name="c", subcore_axis_name="s",
                                    num_cores=NC, num_subcores=NS),
        scratch_types=[
            pltpu.VMEM((NWIN, W), jnp.int32),
            pltpu.VMEM((W, CW), jnp.float32),
            pltpu.VMEM_SHARED((NACC, CW), jnp.float32),
        ],
    )


def _cnt_kernel(dstw, ones, zeros16):
    return _get_cnt_kernel()(dstw, ones, zeros16)


def _pre_body(x_ref, wcat_ref, cnt_ref, u_ref, r_ref, invc_ref):
    ur = jnp.dot(x_ref[...], wcat_ref[...], preferred_element_type=jnp.float32)
    u_ref[...] = ur[:, :H]
    r_ref[...] = ur[:, H:]
    cnt = cnt_ref[0, :N, 0:1] + cnt_ref[1, :N, 0:1]
    invc_ref[...] = 1.0 / jnp.maximum(cnt, 1.0)


def _pre(x, wcat, cnt16):
    return pl.pallas_call(
        _pre_body,
        out_shape=(
            jax.ShapeDtypeStruct((N, H), jnp.float32),
            jax.ShapeDtypeStruct((N, H), jnp.float32),
            jax.ShapeDtypeStruct((N, 1), jnp.float32),
        ),
    )(x, wcat, cnt16)


def _combine_body(has_bn, s_ref, r_ref, invc_ref, bl_ref, bn_ref, wcat_ref,
                  u_ref, rn_ref):
    agg = (s_ref[0, :N, :] + s_ref[1, :N, :]) * invc_ref[...]
    z = agg + bl_ref[...] + r_ref[...]
    if has_bn:
        mu = jnp.mean(z, axis=0, keepdims=True)
        var = jnp.mean((z - mu) * (z - mu), axis=0, keepdims=True)
        z = (z - mu) * lax.rsqrt(var + 1e-5) * bn_ref[0:1, :] + bn_ref[1:2, :]
    h = jnp.maximum(z, 0.0)
    ur = jnp.dot(h, wcat_ref[...], preferred_element_type=jnp.float32)
    u_ref[...] = ur[:, :H]
    rn_ref[...] = ur[:, H:]


def _combine(s, r, invc, bl, bn, wcat, has_bn):
    return pl.pallas_call(
        functools.partial(_combine_body, has_bn),
        out_shape=(
            jax.ShapeDtypeStruct((N, H), jnp.float32),
            jax.ShapeDtypeStruct((N, H), jnp.float32),
        ),
    )(s, r, invc, bl, bn, wcat)


def _head_body(s_ref, r_ref, invc_ref, bl_ref, bn_ref,
               w0_ref, b0_ref, w1_ref, b1_ref, w2_ref, b2_ref, out_ref):
    agg = (s_ref[0, :N, :] + s_ref[1, :N, :]) * invc_ref[...]
    z = agg + bl_ref[...] + r_ref[...]
    mu = jnp.mean(z, axis=0, keepdims=True)
    var = jnp.mean((z - mu) * (z - mu), axis=0, keepdims=True)
    z = (z - mu) * lax.rsqrt(var + 1e-5) * bn_ref[0:1, :] + bn_ref[1:2, :]
    h = jnp.maximum(z, 0.0)
    xm = jnp.mean(h, axis=0, keepdims=True)
    xmx = jnp.max(h, axis=0, keepdims=True)
    g = jnp.concatenate([xm, xmx], axis=1)
    g = jnp.maximum(
        jnp.dot(g, w0_ref[...], preferred_element_type=jnp.float32)
        + b0_ref[...], 0.0)
    g = jnp.maximum(
        jnp.dot(g, w1_ref[...], preferred_element_type=jnp.float32)
        + b1_ref[...], 0.0)
    out_ref[...] = (jnp.dot(g, w2_ref[...], preferred_element_type=jnp.float32)
                    + b2_ref[...])


def _head(s, r, invc, bl, bn, cls):
    return pl.pallas_call(
        _head_body,
        out_shape=jax.ShapeDtypeStruct((1, 1), jnp.float32),
    )(s, r, invc, bl, bn,
      cls[0]["W"].T, cls[0]["b"][None, :],
      cls[1]["W"].T, cls[1]["b"][None, :],
      cls[2]["W"].T, cls[2]["b"][None, :])


def kernel(x, edge_index, params):
    src = edge_index[0].reshape(NC, NS, PER_TILE)
    dst = edge_index[1].reshape(NC, NS, PER_TILE)
    src = jnp.pad(src, ((0, 0), (0, 0), (0, P - PER_TILE)))
    dst = jnp.pad(dst, ((0, 0), (0, 0), (0, P - PER_TILE)),
                  constant_values=TRASH)
    srcw = src.reshape(NC, NS, NWIN, W)
    dstw = dst.reshape(NC, NS, NWIN, W)

    zeros = jnp.zeros((NACC, H), jnp.float32)
    zeros16 = jnp.zeros((NACC, CW), jnp.float32)
    ones = jnp.ones((W, CW), jnp.float32)

    convs = params["convs"]
    wcats, bls = [], []
    for i in range(L):
        for lin in (convs[i]["l1"], convs[i]["l2"]):
            wcats.append(jnp.concatenate([lin["Wl"].T, lin["Wr"].T], axis=1))
            bls.append(lin["bl"][None, :])
    bns = [jnp.stack([params["bns"][i]["g"], params["bns"][i]["b"]])
           for i in range(L)]

    cnt16 = _cnt_kernel(dstw, ones, zeros16)
    u, r, invc = _pre(x, wcats[0], cnt16)
    for i in range(7):
        s = _seg_kernel(u, srcw, dstw, zeros)
        has_bn = (i % 2) == 1
        bn = bns[i // 2] if has_bn else bns[0]
        u, r = _combine(s, r, invc, bls[i], bn, wcats[i + 1], has_bn)
    s = _seg_kernel(u, srcw, dstw, zeros)
    return _head(s, r, invc, bls[7], bns[3], params["cls"])
